# baseline (speedup 1.0000x reference)
import jax
import jax.numpy as jnp
from jax import lax
from jax.experimental import pallas as pl
from jax.experimental.pallas import tpu as pltpu
from jax.experimental.pallas import tpu_sc as plsc

_T, _B, _C = 1024, 128, 192
_REMAIN = _T - (_T * 3) // 4
_NC, _NS = 2, 16
_NW = _NC * _NS
_SLABS_PER_W = _REMAIN // _NW
_L = 16


def _body(fwd_hbm, table_hbm, out_hbm, idx_hbm, fwd_v, sem_i, sem_g):
    wid = lax.axis_index("s") * _NC + lax.axis_index("c")
    t0 = wid * _SLABS_PER_W
    idx_cp = pltpu.async_copy(fwd_hbm.at[pl.ds(t0, _SLABS_PER_W)],
                              idx_hbm.at[pl.ds(t0, _SLABS_PER_W)], sem_i)
    pltpu.sync_copy(fwd_hbm.at[pl.ds(t0, _SLABS_PER_W)], fwd_v)
    lanes = lax.iota(jnp.int32, _L)

    @pl.loop(0, _SLABS_PER_W)
    def _slab(j):
        t = t0 + j
        handles = []
        for i in range(_B // _L):
            vec = fwd_v[j, pl.ds(i * _L, _L)]
            for k in range(_L):
                row = vec[k]
                b = i * _L + k
                handles.append(
                    pltpu.async_copy(table_hbm.at[row, b], out_hbm.at[t, b],
                                     sem_g))
        for h in handles:
            h.wait()

    idx_cp.wait()


@jax.jit
def _shuffle(fwd, table):
    mesh = plsc.VectorSubcoreMesh(core_axis_name="c", subcore_axis_name="s")
    out, idx = pl.kernel(
        _body,
        out_type=(
            jax.ShapeDtypeStruct((_REMAIN, _B, _C), jnp.float32),
            jax.ShapeDtypeStruct((_REMAIN, _B), jnp.int32),
        ),
        mesh=mesh,
        scratch_types=[
            pltpu.VMEM((_SLABS_PER_W, _B), jnp.int32),
            pltpu.SemaphoreType.DMA,
            pltpu.SemaphoreType.DMA,
        ],
    )(fwd, table)
    return out, idx


def kernel(patches, forward_indexes):
    return _shuffle(forward_indexes, patches)


# speedup vs baseline: 3.0589x; 3.0589x over previous
import jax
import jax.numpy as jnp
from jax import lax
from jax.experimental import pallas as pl
from jax.experimental.pallas import tpu as pltpu
from jax.experimental.pallas import tpu_sc as plsc

_T, _B, _C = 1024, 128, 192
_REMAIN = _T - (_T * 3) // 4
_NC, _NS = 2, 16
_NW = _NC * _NS
_SLABS_PER_W = _T // _NW
_OSLABS_PER_W = _REMAIN // _NW
_NBUF = 4
_L = 16
_SP = _B + 1


def _tr_body(src_hbm, tbl_hbm, in_bufs, tr_bufs, sem_i, sem_o):
    wid = lax.axis_index("s") * _NC + lax.axis_index("c")
    s0 = wid * _SLABS_PER_W
    rows16 = [lax.iota(jnp.int32, _L) + m * _L for m in range(_C // _L)]

    def in_cp(j, k):
        return pltpu.make_async_copy(
            src_hbm.at[pl.ds((s0 + j) * _C, _C)],
            in_bufs[k].at[:, pl.ds(0, _B)], sem_i[k])

    def out_cp(j, k):
        return pltpu.make_async_copy(
            tr_bufs[k], tbl_hbm.at[pl.ds((s0 + j) * _B, _B)], sem_o[k])

    def transpose(k):
        @pl.loop(0, _B)
        def _col(b):
            col = jnp.full((_L,), b, dtype=jnp.int32)
            for m in range(_C // _L):
                v = plsc.load_gather(in_bufs[k], [rows16[m], col])
                tr_bufs[k][b, pl.ds(m * _L, _L)] = v

    in_cp(0, 0).start()

    @pl.loop(0, _SLABS_PER_W, step=2)
    def _pair(s):
        in_cp(s + 1, 1).start()
        in_cp(s, 0).wait()

        @pl.when(s >= 2)
        def _():
            out_cp(s - 2, 0).wait()

        transpose(0)
        out_cp(s, 0).start()

        @pl.when(s + 2 < _SLABS_PER_W)
        def _():
            in_cp(s + 2, 0).start()

        in_cp(s + 1, 1).wait()

        @pl.when(s >= 2)
        def _():
            out_cp(s - 1, 1).wait()

        transpose(1)
        out_cp(s + 1, 1).start()

    out_cp(_SLABS_PER_W - 2, 0).wait()
    out_cp(_SLABS_PER_W - 1, 1).wait()


def _gather_body(fwd_hbm, tbl_hbm, out_hbm, idx_hbm, raw_v, g_v, rows, sem_i,
                 sem_g, sem_s):
    wid = lax.axis_index("s") * _NC + lax.axis_index("c")
    t0 = wid * _OSLABS_PER_W
    lanes = lax.iota(jnp.int32, _L)

    pltpu.sync_copy(fwd_hbm.at[pl.ds(t0, _OSLABS_PER_W)], raw_v)
    idx_cp = pltpu.async_copy(raw_v, idx_hbm.at[pl.ds(t0, _OSLABS_PER_W)],
                              sem_i)
    for j in range(_OSLABS_PER_W):
        for i in range(_B // _L):
            g_v[j, pl.ds(i * _L, _L)] = (
                raw_v[j, pl.ds(i * _L, _L)] * _B + lanes + i * _L
            )

    def start_gather(j):
        return pltpu.async_copy(tbl_hbm.at[g_v.at[j]], rows[j % _NBUF],
                                sem_g[j % _NBUF])

    gathers = {j: start_gather(j) for j in range(_NBUF)}
    stores = {}
    for j in range(_OSLABS_PER_W):
        if j >= 1 and j + _NBUF - 1 < _OSLABS_PER_W:
            stores[j - 1].wait()
            gathers[j + _NBUF - 1] = start_gather(j + _NBUF - 1)
        gathers[j].wait()
        stores[j] = pltpu.async_copy(
            rows[j % _NBUF],
            out_hbm.at[pl.ds((t0 + j) * _B, _B)],
            sem_s[j % _NBUF])
    for j in range(_OSLABS_PER_W - min(_NBUF, _OSLABS_PER_W), _OSLABS_PER_W):
        stores[j].wait()
    idx_cp.wait()


@jax.jit
def _shuffle(fwd, src):
    mesh = plsc.VectorSubcoreMesh(core_axis_name="c", subcore_axis_name="s")
    table = pl.kernel(
        _tr_body,
        out_type=jax.ShapeDtypeStruct((_T * _B, _C), jnp.float32),
        mesh=mesh,
        compiler_params=pltpu.CompilerParams(use_tc_tiling_on_sc=False,
                                             needs_layout_passes=False),
        scratch_types=[
            [pltpu.VMEM((_C, _SP), jnp.float32) for _ in range(2)],
            [pltpu.VMEM((_B, _C), jnp.float32) for _ in range(2)],
            [pltpu.SemaphoreType.DMA for _ in range(2)],
            [pltpu.SemaphoreType.DMA for _ in range(2)],
        ],
    )(src)
    out, idx = pl.kernel(
        _gather_body,
        out_type=(
            jax.ShapeDtypeStruct((_REMAIN * _B, _C), jnp.float32),
            jax.ShapeDtypeStruct((_REMAIN, _B), jnp.int32),
        ),
        mesh=mesh,
        compiler_params=pltpu.CompilerParams(use_tc_tiling_on_sc=False),
        scratch_types=[
            pltpu.VMEM((_OSLABS_PER_W, _B), jnp.int32),
            pltpu.VMEM((_OSLABS_PER_W, _B), jnp.int32),
            [pltpu.VMEM((_B, _C), jnp.float32) for _ in range(_NBUF)],
            pltpu.SemaphoreType.DMA,
            [pltpu.SemaphoreType.DMA for _ in range(_NBUF)],
            [pltpu.SemaphoreType.DMA for _ in range(_NBUF)],
        ],
    )(fwd, table)
    return out, idx


def kernel(patches, forward_indexes):
    src = jnp.swapaxes(patches, 1, 2).reshape(_T * _C, _B)
    fwd = forward_indexes.reshape(_T, _B)
    out, idx = _shuffle(fwd, src)
    return out.reshape(_REMAIN, _B, _C), idx
